# baseline (speedup 1.0000x reference)
import functools

import jax
import jax.numpy as jnp
from jax import lax
from jax.experimental import pallas as pl
from jax.experimental.pallas import tpu as pltpu
from jax.experimental.pallas import tpu_sc as plsc

EMBED = 64
IMG_TOK = 151669
BATCH = 16
SEQ = 2048
TOT = BATCH * SEQ

NC, NS = 2, 16
NW = NC * NS
ROWS_PER_W = TOT // NW
IDX_CHUNK = 128
CH = ROWS_PER_W // IDX_CHUNK

@functools.cache
def _build_sc_gather():
    mesh = plsc.VectorSubcoreMesh(
        core_axis_name="c", subcore_axis_name="s", num_cores=NC, num_subcores=NS
    )

    @functools.partial(
        pl.kernel,
        mesh=mesh,
        out_type=jax.ShapeDtypeStruct((TOT, EMBED), jnp.float32),
        scratch_types=[
            pltpu.VMEM((CH, IDX_CHUNK), jnp.int32),
            pltpu.VMEM((ROWS_PER_W, EMBED), jnp.float32),
            pltpu.SemaphoreType.DMA,
        ],
        compiler_params=pltpu.CompilerParams(use_tc_tiling_on_sc=False),
    )
    def _sc_gather(table_hbm, ids_hbm, out_hbm, idx_v, rows_v, sem):
        wid = lax.axis_index("s") * NC + lax.axis_index("c")
        pltpu.sync_copy(ids_hbm.at[pl.ds(wid * CH, CH)], idx_v)
        copies = []
        for j in range(CH):
            copies.append(
                pltpu.async_copy(
                    table_hbm.at[idx_v.at[j]],
                    rows_v.at[pl.ds(j * IDX_CHUNK, IDX_CHUNK)],
                    sem,
                )
            )
        for c in copies:
            c.wait()
        pltpu.sync_copy(rows_v, out_hbm.at[pl.ds(wid * ROWS_PER_W, ROWS_PER_W)])

    return _sc_gather


def _mask_body(ids_ref, attn_ref, am_out, im_out):
    am_out[...] = attn_ref[...] == 1
    im_out[...] = ids_ref[...] == IMG_TOK


def _masks_tc(input_ids, attention_mask):
    return pl.pallas_call(
        _mask_body,
        out_shape=(
            jax.ShapeDtypeStruct((BATCH, SEQ), jnp.bool_),
            jax.ShapeDtypeStruct((BATCH, SEQ), jnp.bool_),
        ),
    )(input_ids, attention_mask)


def kernel(pixel_values, input_ids, attention_mask, text_proj_weight):
    del pixel_values
    ids32 = input_ids.astype(jnp.int32)
    ids_tiled = ids32.reshape(TOT // IDX_CHUNK, IDX_CHUNK)
    flat = _build_sc_gather()(text_proj_weight, ids_tiled)
    hidden_states = flat.reshape(BATCH, SEQ, EMBED)
    attn_mask, image_mask = _masks_tc(ids32, attention_mask.astype(jnp.int32))
    return (hidden_states, attn_mask, image_mask)
